# Initial kernel scaffold; baseline (speedup 1.0000x reference)
import jax
import jax.numpy as jnp
from jax.experimental import pallas as pl


def kernel(input_a, input_b, in_channels_a, out_channels_a, in_channels_b, out_channels_b):
    raise NotImplementedError("write your pallas kernel here")


# speedup vs baseline: 2.8766x; 2.8766x over previous
import jax
import jax.numpy as jnp
from jax.experimental import pallas as pl
from jax.experimental.pallas import tpu as pltpu


def _add_body(src_a_ref, src_b_ref, val_a_ref, val_b_ref, a_ref, b_ref, o_ref):
    c = pl.program_id(1)
    va = val_a_ref[c].astype(jnp.float32)
    vb = val_b_ref[c].astype(jnp.float32)
    o_ref[...] = a_ref[...] * va + b_ref[...] * vb


def kernel(input_a, input_b, in_channels_a, out_channels_a, in_channels_b, out_channels_b):
    B, C, H, W = input_a.shape

    ins_a = in_channels_a.astype(jnp.int32)
    outs_a = out_channels_a.astype(jnp.int32)
    ins_b = in_channels_b.astype(jnp.int32)
    outs_b = out_channels_b.astype(jnp.int32)
    src_a = jnp.zeros((C,), jnp.int32).at[outs_a].set(ins_a)
    val_a = jnp.zeros((C,), jnp.int32).at[outs_a].set(1)
    src_b = jnp.zeros((C,), jnp.int32).at[outs_b].set(ins_b)
    val_b = jnp.zeros((C,), jnp.int32).at[outs_b].set(1)

    grid_spec = pltpu.PrefetchScalarGridSpec(
        num_scalar_prefetch=4,
        grid=(B, C),
        in_specs=[
            pl.BlockSpec((1, 1, H, W), lambda b, c, sa, sb, va, vb: (b, sa[c], 0, 0)),
            pl.BlockSpec((1, 1, H, W), lambda b, c, sa, sb, va, vb: (b, sb[c], 0, 0)),
        ],
        out_specs=pl.BlockSpec((1, 1, H, W), lambda b, c, sa, sb, va, vb: (b, c, 0, 0)),
    )

    return pl.pallas_call(
        _add_body,
        grid_spec=grid_spec,
        out_shape=jax.ShapeDtypeStruct((B, C, H, W), input_a.dtype),
    )(src_a, src_b, val_a, val_b, input_a, input_b)


# speedup vs baseline: 6.5405x; 2.2737x over previous
import jax
import jax.numpy as jnp
from jax.experimental import pallas as pl
from jax.experimental.pallas import tpu as pltpu


def _add_body(src_a_ref, src_b_ref, val_a_ref, val_b_ref, a_ref, b_ref, o_ref):
    c = pl.program_id(0)
    va = val_a_ref[c].astype(jnp.float32)
    vb = val_b_ref[c].astype(jnp.float32)
    o_ref[...] = a_ref[...] * va + b_ref[...] * vb


def kernel(input_a, input_b, in_channels_a, out_channels_a, in_channels_b, out_channels_b):
    B, C, H, W = input_a.shape

    ins_a = in_channels_a.astype(jnp.int32)
    outs_a = out_channels_a.astype(jnp.int32)
    ins_b = in_channels_b.astype(jnp.int32)
    outs_b = out_channels_b.astype(jnp.int32)
    src_a = jnp.zeros((C,), jnp.int32).at[outs_a].set(ins_a)
    val_a = jnp.zeros((C,), jnp.int32).at[outs_a].set(1)
    src_b = jnp.zeros((C,), jnp.int32).at[outs_b].set(ins_b)
    val_b = jnp.zeros((C,), jnp.int32).at[outs_b].set(1)

    grid_spec = pltpu.PrefetchScalarGridSpec(
        num_scalar_prefetch=4,
        grid=(C,),
        in_specs=[
            pl.BlockSpec((B, 1, H, W), lambda c, sa, sb, va, vb: (0, sa[c], 0, 0)),
            pl.BlockSpec((B, 1, H, W), lambda c, sa, sb, va, vb: (0, sb[c], 0, 0)),
        ],
        out_specs=pl.BlockSpec((B, 1, H, W), lambda c, sa, sb, va, vb: (0, c, 0, 0)),
    )

    return pl.pallas_call(
        _add_body,
        grid_spec=grid_spec,
        out_shape=jax.ShapeDtypeStruct((B, C, H, W), input_a.dtype),
    )(src_a, src_b, val_a, val_b, input_a, input_b)
